# baseline (speedup 1.0000x reference)
import functools

import jax
import jax.numpy as jnp
from jax import lax
from jax.experimental import pallas as pl
from jax.experimental.pallas import tpu as pltpu
from jax.experimental.pallas import tpu_sc as plsc

D = 64
SCALE = 8.0
B = 4096 * 200
NC = 2
NS = 16
NW = NC * NS
BPW = B // NW
C = 128
NCHUNK = BPW // C


def _sc_gather(idx3, table):
    mesh = plsc.VectorSubcoreMesh(core_axis_name="c", subcore_axis_name="s")

    @functools.partial(
        pl.kernel,
        mesh=mesh,
        out_type=jax.ShapeDtypeStruct((B, D), jnp.float32),
        scratch_types=[
            pltpu.VMEM((NCHUNK, C), jnp.int32),
            pltpu.VMEM((C, D), jnp.float32),
            pltpu.SemaphoreType.DMA,
        ],
        compiler_params=pltpu.CompilerParams(use_tc_tiling_on_sc=False),
    )
    def kern(idx_hbm, table_hbm, out_hbm, idx_v, rows_v, sem):
        wid = lax.axis_index("s") * NC + lax.axis_index("c")
        base = wid * BPW
        pltpu.sync_copy(idx_hbm.at[wid], idx_v)

        def body(ci, _):
            pltpu.async_copy(table_hbm.at[idx_v.at[ci]], rows_v, sem).wait()

            def srow(r, _):
                for c in range(D // 16):
                    sl = pl.ds(c * 16, 16)
                    rows_v[r, sl] = rows_v[r, sl] * SCALE
                return 0

            lax.fori_loop(0, C, srow, 0)
            pltpu.sync_copy(rows_v, out_hbm.at[pl.ds(base + ci * C, C)])
            return 0

        lax.fori_loop(0, NCHUNK, body, 0)

    return kern(idx3, table)


def kernel(inp, emb_weight):
    idx3 = inp.reshape(NW, NCHUNK, C)
    out = _sc_gather(idx3, emb_weight)
    return out.reshape(4096, 200, D)
